# Initial kernel scaffold; baseline (speedup 1.0000x reference)
import jax
import jax.numpy as jnp
from jax.experimental import pallas as pl


def kernel(x, edge_index, batch, W_l, b_l, W_r, W2, b2):
    raise NotImplementedError("write your pallas kernel here")


# speedup vs baseline: 9.2686x; 9.2686x over previous
import functools

import jax
import jax.numpy as jnp
from jax import lax
from jax.experimental import pallas as pl
from jax.experimental.pallas import tpu as pltpu
from jax.experimental.pallas import tpu_sc as plsc

N = 10000
E = 320000
D = 128
H = 128
G = 64

CHUNK = 128
NW = 32
CH_PER_W = 80
NCH = E // CHUNK
E_PAD = CH_PER_W * NW * CHUNK
N_PAD = 10112
ZROWS = N_PAD // 16
G_PER_W = E_PAD // 16 // NW
PAD_DST = 10008

_mesh = plsc.VectorSubcoreMesh(core_axis_name="c", subcore_axis_name="s")


@functools.partial(
    pl.kernel,
    mesh=_mesh,
    out_type=jax.ShapeDtypeStruct((2, N_PAD, D), jnp.float32),
    scratch_types=[
        pltpu.VMEM((CH_PER_W, CHUNK), jnp.int32),
        pltpu.VMEM((CH_PER_W, CHUNK), jnp.int32),
        pltpu.VMEM((CHUNK, D), jnp.float32),
        pltpu.VMEM_SHARED((N_PAD, D), jnp.float32),
        pltpu.SemaphoreType.DMA,
    ],
)
def _sc_aggregate(edge_hbm, x_hbm, zrow_hbm, agg_out,
                  sidx, didx, rows, acc_sh, sem):
    c = lax.axis_index("c")
    s = lax.axis_index("s")
    w = c * 16 + s

    z0 = s * ZROWS
    pltpu.sync_copy(zrow_hbm.at[pl.ds(z0, ZROWS)], acc_sh.at[pl.ds(z0, ZROWS)])

    c0 = w * CH_PER_W
    pltpu.sync_copy(edge_hbm.at[0, pl.ds(c0, CH_PER_W)], sidx)
    pltpu.sync_copy(edge_hbm.at[1, pl.ds(c0, CH_PER_W)], didx)
    plsc.subcore_barrier()

    def chunk_body(j, carry):
        @pl.when(c0 + j < NCH)
        def _():
            pltpu.async_copy(x_hbm.at[sidx.at[j]], rows, sem).wait()
            pltpu.sync_copy(rows, acc_sh.at[didx.at[j]], add=True)
        return carry

    lax.fori_loop(0, CH_PER_W, chunk_body, 0)
    plsc.subcore_barrier()

    pltpu.sync_copy(acc_sh.at[pl.ds(z0, ZROWS)], agg_out.at[c, pl.ds(z0, ZROWS)])


@functools.partial(
    pl.kernel,
    mesh=_mesh,
    out_type=jax.ShapeDtypeStruct((2, 16, N_PAD), jnp.float32),
    scratch_types=[
        pltpu.VMEM((G_PER_W, 16), jnp.int32),
        pltpu.VMEM((N_PAD,), jnp.float32),
    ],
    compiler_params=pltpu.CompilerParams(needs_layout_passes=False),
)
def _sc_degree(dst16_hbm, deg_out, didx16, deg_l):
    c = lax.axis_index("c")
    s = lax.axis_index("s")
    w = c * 16 + s

    def _dz_body(i, carry):
        deg_l[pl.ds(i * 16, 16)] = jnp.zeros((16,), jnp.float32)
        return carry

    lax.fori_loop(0, N_PAD // 16, _dz_body, 0)

    pltpu.sync_copy(dst16_hbm.at[pl.ds(w * G_PER_W, G_PER_W)], didx16)

    ones = jnp.full((16,), 1.0, jnp.float32)

    def g_body(i, carry):
        plsc.addupdate_scatter(deg_l, [didx16[i]], ones)
        return carry

    lax.fori_loop(0, G_PER_W, g_body, 0)
    pltpu.sync_copy(deg_l, deg_out.at[c, s])


BN = 1000
NBLK = N // BN


def _tc_body(x_ref, a_ref, d_ref, b_ref, wl_ref, wr_ref, bl_ref, w2_ref,
             b2_ref, out_ref, pooled):
    pid = pl.program_id(0)

    @pl.when(pid == 0)
    def _():
        pooled[...] = jnp.full((G, H), -jnp.inf, jnp.float32)

    deg = jnp.sum(d_ref[...], axis=1, keepdims=True)
    mean = (a_ref[0] + a_ref[1]) / jnp.maximum(deg, 1.0)
    h = (jnp.dot(mean, wl_ref[...], preferred_element_type=jnp.float32)
         + jnp.dot(x_ref[...], wr_ref[...], preferred_element_type=jnp.float32)
         + bl_ref[...])
    h = jnp.maximum(h, 0.0)

    g0 = b_ref[0, 0]
    g1 = b_ref[BN - 1, 0]

    def gbody(g, carry):
        pen = jnp.where(b_ref[...] == g, 0.0, -jnp.inf)
        m = jnp.max(h + pen, axis=0, keepdims=True)
        pooled[pl.ds(g, 1), :] = jnp.maximum(pooled[pl.ds(g, 1), :], m)
        return carry

    lax.fori_loop(g0, g1 + 1, gbody, 0)

    @pl.when(pid == NBLK - 1)
    def _():
        pf = pooled[...]
        pf = jnp.where(jnp.isfinite(pf), pf, 0.0)
        out_ref[...] = (
            jnp.dot(pf, w2_ref[...], preferred_element_type=jnp.float32)
            + b2_ref[...])


_tc_call = pl.pallas_call(
    _tc_body,
    grid=(NBLK,),
    in_specs=[
        pl.BlockSpec((BN, D), lambda i: (i, 0)),
        pl.BlockSpec((2, BN, D), lambda i: (0, i, 0)),
        pl.BlockSpec((BN, NW), lambda i: (i, 0)),
        pl.BlockSpec((BN, 1), lambda i: (i, 0)),
        pl.BlockSpec((D, H), lambda i: (0, 0)),
        pl.BlockSpec((D, H), lambda i: (0, 0)),
        pl.BlockSpec((1, H), lambda i: (0, 0)),
        pl.BlockSpec((H, 1), lambda i: (0, 0)),
        pl.BlockSpec((1, 1), lambda i: (0, 0)),
    ],
    out_specs=pl.BlockSpec((G, 1), lambda i: (0, 0)),
    out_shape=jax.ShapeDtypeStruct((G, 1), jnp.float32),
    scratch_shapes=[pltpu.VMEM((G, H), jnp.float32)],
)


def kernel(x, edge_index, batch, W_l, b_l, W_r, W2, b2):
    pad = E_PAD - E
    edge_p = jnp.concatenate(
        [edge_index,
         jnp.stack([jnp.zeros((pad,), jnp.int32),
                    jnp.full((pad,), PAD_DST, jnp.int32)])], axis=1)
    dst16 = edge_p[1].reshape(E_PAD // 16, 16)
    edge_p = edge_p.reshape(2, E_PAD // CHUNK, CHUNK)

    zrow = jnp.zeros((N_PAD, D), jnp.float32)
    agg2 = _sc_aggregate(edge_p, x, zrow)
    deg2 = _sc_degree(dst16)

    degT = jnp.transpose(deg2, (2, 0, 1)).reshape(N_PAD, NW)
    out = _tc_call(x, agg2, degT, batch.reshape(N, 1), W_l, W_r,
                   b_l.reshape(1, H), W2, b2.reshape(1, 1))
    return out.reshape(-1)


# speedup vs baseline: 11.2490x; 1.2137x over previous
import functools

import jax
import jax.numpy as jnp
from jax import lax
from jax.experimental import pallas as pl
from jax.experimental.pallas import tpu as pltpu
from jax.experimental.pallas import tpu_sc as plsc

N = 10000
E = 320000
D = 128
H = 128
G = 64

CHUNK = 128
NW = 32
CH_PER_W = 80
NCH = E // CHUNK
E_PAD = CH_PER_W * NW * CHUNK
N_PAD = 10112
ZROWS = N_PAD // 16
G_PER_W = E_PAD // 16 // NW
PAD_DST = 10008

_mesh = plsc.VectorSubcoreMesh(core_axis_name="c", subcore_axis_name="s")


@functools.partial(
    pl.kernel,
    mesh=_mesh,
    out_type=jax.ShapeDtypeStruct((2, N_PAD, D), jnp.float32),
    scratch_types=[
        pltpu.VMEM((CH_PER_W // 2, CHUNK), jnp.int32),
        pltpu.VMEM((CH_PER_W // 2, CHUNK), jnp.int32),
        pltpu.VMEM((2, CHUNK, D), jnp.float32),
        pltpu.VMEM_SHARED((N_PAD, D), jnp.float32),
        pltpu.SemaphoreType.DMA,
    ],
)
def _sc_aggregate(edge_hbm, x_hbm, agg_out,
                  sidx, didx, rows, acc_sh, gsem):
    c = lax.axis_index("c")
    s = lax.axis_index("s")
    w = c * 16 + s

    def _z_body(i, carry):
        rows[0, i // 8, pl.ds((i % 8) * 16, 16)] = jnp.zeros((16,), jnp.float32)
        return carry

    lax.fori_loop(0, CHUNK * 8, _z_body, 0)
    z0 = s * ZROWS
    for t in range(4):
        pltpu.sync_copy(rows.at[0],
                        acc_sh.at[pl.ds(z0 + t * CHUNK, CHUNK)])
    pltpu.sync_copy(rows.at[0, pl.ds(0, ZROWS - 4 * CHUNK)],
                    acc_sh.at[pl.ds(z0 + 4 * CHUNK, ZROWS - 4 * CHUNK)])

    plsc.subcore_barrier()

    c0 = w * CH_PER_W
    nv = jnp.minimum(CH_PER_W, NCH - c0)
    HALF = CH_PER_W // 2

    for p in range(2):
        nvp = jnp.clip(nv - p * HALF, 0, HALF)

        @pl.when(nvp > 0)
        def _():
            pltpu.sync_copy(edge_hbm.at[0, pl.ds(c0 + p * HALF, HALF)], sidx)
            pltpu.sync_copy(edge_hbm.at[1, pl.ds(c0 + p * HALF, HALF)], didx)
            pltpu.async_copy(x_hbm.at[sidx.at[0]], rows.at[0], gsem)

            def chunk_body(i, carry):
                b = i % 2
                nb = 1 - b

                pltpu.make_async_copy(
                    x_hbm.at[pl.ds(0, CHUNK)], rows.at[b], gsem).wait()

                @pl.when(i + 1 < nvp)
                def _():
                    pltpu.async_copy(
                        x_hbm.at[sidx.at[i + 1]], rows.at[nb], gsem)

                pltpu.sync_copy(rows.at[b], acc_sh.at[didx.at[i]], add=True)
                return carry

            lax.fori_loop(0, nvp, chunk_body, 0)

    plsc.subcore_barrier()

    pltpu.sync_copy(acc_sh.at[pl.ds(z0, ZROWS)], agg_out.at[c, pl.ds(z0, ZROWS)])


@functools.partial(
    pl.kernel,
    mesh=_mesh,
    out_type=jax.ShapeDtypeStruct((2, 16, N_PAD), jnp.float32),
    scratch_types=[
        pltpu.VMEM((G_PER_W, 16), jnp.int32),
        pltpu.VMEM((N_PAD,), jnp.float32),
    ],
    compiler_params=pltpu.CompilerParams(needs_layout_passes=False),
)
def _sc_degree(dst16_hbm, deg_out, didx16, deg_l):
    c = lax.axis_index("c")
    s = lax.axis_index("s")
    w = c * 16 + s

    def _dz_body(i, carry):
        deg_l[pl.ds(i * 16, 16)] = jnp.zeros((16,), jnp.float32)
        return carry

    lax.fori_loop(0, N_PAD // 16, _dz_body, 0)

    pltpu.sync_copy(dst16_hbm.at[pl.ds(w * G_PER_W, G_PER_W)], didx16)

    ones = jnp.full((16,), 1.0, jnp.float32)

    def g_body(i, carry):
        plsc.addupdate_scatter(deg_l, [didx16[i]], ones)
        return carry

    lax.fori_loop(0, G_PER_W, g_body, 0)
    pltpu.sync_copy(deg_l, deg_out.at[c, s])


BN = 1000
NBLK = N // BN


def _tc_body(x_ref, a_ref, d_ref, b_ref, wl_ref, wr_ref, bl_ref, w2_ref,
             b2_ref, out_ref, pooled):
    pid = pl.program_id(0)

    @pl.when(pid == 0)
    def _():
        pooled[...] = jnp.full((G, H), -jnp.inf, jnp.float32)

    deg = jnp.sum(d_ref[...], axis=1, keepdims=True)
    mean = (a_ref[0] + a_ref[1]) / jnp.maximum(deg, 1.0)
    h = (jnp.dot(mean, wl_ref[...], preferred_element_type=jnp.float32)
         + jnp.dot(x_ref[...], wr_ref[...], preferred_element_type=jnp.float32)
         + bl_ref[...])
    h = jnp.maximum(h, 0.0)

    g0 = b_ref[0, 0]
    g1 = b_ref[BN - 1, 0]

    def gbody(g, carry):
        pen = jnp.where(b_ref[...] == g, 0.0, -jnp.inf)
        m = jnp.max(h + pen, axis=0, keepdims=True)
        pooled[pl.ds(g, 1), :] = jnp.maximum(pooled[pl.ds(g, 1), :], m)
        return carry

    lax.fori_loop(g0, g1 + 1, gbody, 0)

    @pl.when(pid == NBLK - 1)
    def _():
        pf = pooled[...]
        pf = jnp.where(jnp.isfinite(pf), pf, 0.0)
        out_ref[...] = (
            jnp.dot(pf, w2_ref[...], preferred_element_type=jnp.float32)
            + b2_ref[...])


_tc_call = pl.pallas_call(
    _tc_body,
    grid=(NBLK,),
    in_specs=[
        pl.BlockSpec((BN, D), lambda i: (i, 0)),
        pl.BlockSpec((2, BN, D), lambda i: (0, i, 0)),
        pl.BlockSpec((BN, NW), lambda i: (i, 0)),
        pl.BlockSpec((BN, 1), lambda i: (i, 0)),
        pl.BlockSpec((D, H), lambda i: (0, 0)),
        pl.BlockSpec((D, H), lambda i: (0, 0)),
        pl.BlockSpec((1, H), lambda i: (0, 0)),
        pl.BlockSpec((H, 1), lambda i: (0, 0)),
        pl.BlockSpec((1, 1), lambda i: (0, 0)),
    ],
    out_specs=pl.BlockSpec((G, 1), lambda i: (0, 0)),
    out_shape=jax.ShapeDtypeStruct((G, 1), jnp.float32),
    scratch_shapes=[pltpu.VMEM((G, H), jnp.float32)],
)


def kernel(x, edge_index, batch, W_l, b_l, W_r, W2, b2):
    pad = E_PAD - E
    edge_p = jnp.concatenate(
        [edge_index,
         jnp.stack([jnp.zeros((pad,), jnp.int32),
                    jnp.full((pad,), PAD_DST, jnp.int32)])], axis=1)
    dst16 = edge_p[1].reshape(E_PAD // 16, 16)
    edge_p = edge_p.reshape(2, E_PAD // CHUNK, CHUNK)

    agg2 = _sc_aggregate(edge_p, x)
    deg2 = _sc_degree(dst16)

    degT = jnp.transpose(deg2, (2, 0, 1)).reshape(N_PAD, NW)
    out = _tc_call(x, agg2, degT, batch.reshape(N, 1), W_l, W_r,
                   b_l.reshape(1, H), W2, b2.reshape(1, 1))
    return out.reshape(-1)


# speedup vs baseline: 11.3551x; 1.0094x over previous
import functools

import jax
import jax.numpy as jnp
from jax import lax
from jax.experimental import pallas as pl
from jax.experimental.pallas import tpu as pltpu
from jax.experimental.pallas import tpu_sc as plsc

N = 10000
E = 320000
D = 128
H = 128
G = 64

CHUNK = 128
NW = 32
CH_PER_W = 80
NCH = E // CHUNK
E_PAD = CH_PER_W * NW * CHUNK
N_PAD = 10112
ZROWS = N_PAD // 16
G_PER_W = E_PAD // 16 // NW
PAD_DST = 10008

_mesh = plsc.VectorSubcoreMesh(core_axis_name="c", subcore_axis_name="s")


@functools.partial(
    pl.kernel,
    mesh=_mesh,
    out_type=jax.ShapeDtypeStruct((2, N_PAD, D), jnp.float32),
    scratch_types=[
        pltpu.VMEM((CH_PER_W // 2, CHUNK), jnp.int32),
        pltpu.VMEM((CH_PER_W // 2, CHUNK), jnp.int32),
        pltpu.VMEM((2, CHUNK, D), jnp.float32),
        pltpu.VMEM_SHARED((N_PAD, D), jnp.float32),
        pltpu.SemaphoreType.DMA,
        pltpu.SemaphoreType.DMA,
    ],
)
def _sc_aggregate(edge_hbm, x_hbm, agg_out,
                  sidx, didx, rows, acc_sh, gsem, ssem):
    c = lax.axis_index("c")
    s = lax.axis_index("s")
    w = c * 16 + s

    def _z_body(i, carry):
        rows[0, i // 8, pl.ds((i % 8) * 16, 16)] = jnp.zeros((16,), jnp.float32)
        return carry

    lax.fori_loop(0, CHUNK * 8, _z_body, 0)
    z0 = s * ZROWS
    for t in range(4):
        pltpu.sync_copy(rows.at[0],
                        acc_sh.at[pl.ds(z0 + t * CHUNK, CHUNK)])
    pltpu.sync_copy(rows.at[0, pl.ds(0, ZROWS - 4 * CHUNK)],
                    acc_sh.at[pl.ds(z0 + 4 * CHUNK, ZROWS - 4 * CHUNK)])

    plsc.subcore_barrier()

    c0 = w * CH_PER_W
    nv = jnp.minimum(CH_PER_W, NCH - c0)
    HALF = CH_PER_W // 2

    for p in range(2):
        nvp = jnp.clip(nv - p * HALF, 0, HALF)

        @pl.when(nvp > 0)
        def _():
            pltpu.sync_copy(edge_hbm.at[0, pl.ds(c0 + p * HALF, HALF)], sidx)
            pltpu.sync_copy(edge_hbm.at[1, pl.ds(c0 + p * HALF, HALF)], didx)
            pltpu.async_copy(x_hbm.at[sidx.at[0]], rows.at[0], gsem)

            def chunk_body(i, carry):
                b = i % 2
                nb = 1 - b

                pltpu.make_async_copy(
                    x_hbm.at[pl.ds(0, CHUNK)], rows.at[b], gsem).wait()

                @pl.when(i >= 1)
                def _():
                    pltpu.make_async_copy(
                        x_hbm.at[pl.ds(0, CHUNK)], rows.at[nb], ssem).wait()

                @pl.when(i + 1 < nvp)
                def _():
                    pltpu.async_copy(
                        x_hbm.at[sidx.at[i + 1]], rows.at[nb], gsem)

                pltpu.async_copy(
                    rows.at[b], acc_sh.at[didx.at[i]], ssem, add=True)
                return carry

            lax.fori_loop(0, nvp, chunk_body, 0)
            pltpu.make_async_copy(
                x_hbm.at[pl.ds(0, CHUNK)], rows.at[0], ssem).wait()

    plsc.subcore_barrier()

    pltpu.sync_copy(acc_sh.at[pl.ds(z0, ZROWS)], agg_out.at[c, pl.ds(z0, ZROWS)])


@functools.partial(
    pl.kernel,
    mesh=_mesh,
    out_type=jax.ShapeDtypeStruct((2, 16, N_PAD), jnp.float32),
    scratch_types=[
        pltpu.VMEM((G_PER_W, 16), jnp.int32),
        pltpu.VMEM((N_PAD,), jnp.float32),
    ],
    compiler_params=pltpu.CompilerParams(needs_layout_passes=False),
)
def _sc_degree(dst16_hbm, deg_out, didx16, deg_l):
    c = lax.axis_index("c")
    s = lax.axis_index("s")
    w = c * 16 + s

    def _dz_body(i, carry):
        for k in range(8):
            deg_l[pl.ds((i * 8 + k) * 16, 16)] = jnp.zeros((16,), jnp.float32)
        return carry

    lax.fori_loop(0, N_PAD // 128, _dz_body, 0)

    pltpu.sync_copy(dst16_hbm.at[pl.ds(w * G_PER_W, G_PER_W)], didx16)

    ones = jnp.full((16,), 1.0, jnp.float32)

    def g_body(i, carry):
        for k in range(8):
            plsc.addupdate_scatter(deg_l, [didx16[i * 8 + k]], ones)
        return carry

    lax.fori_loop(0, G_PER_W // 8, g_body, 0)
    pltpu.sync_copy(deg_l, deg_out.at[c, s])


BN = 1000
NBLK = N // BN


def _tc_body(x_ref, a_ref, d_ref, b_ref, wl_ref, wr_ref, bl_ref, w2_ref,
             b2_ref, out_ref, pooled):
    pid = pl.program_id(0)

    @pl.when(pid == 0)
    def _():
        pooled[...] = jnp.full((G, H), -jnp.inf, jnp.float32)

    deg = jnp.sum(d_ref[...], axis=1, keepdims=True)
    mean = (a_ref[0] + a_ref[1]) / jnp.maximum(deg, 1.0)
    h = (jnp.dot(mean, wl_ref[...], preferred_element_type=jnp.float32)
         + jnp.dot(x_ref[...], wr_ref[...], preferred_element_type=jnp.float32)
         + bl_ref[...])
    h = jnp.maximum(h, 0.0)

    g0 = b_ref[0, 0]
    g1 = b_ref[BN - 1, 0]

    def gbody(g, carry):
        pen = jnp.where(b_ref[...] == g, 0.0, -jnp.inf)
        m = jnp.max(h + pen, axis=0, keepdims=True)
        pooled[pl.ds(g, 1), :] = jnp.maximum(pooled[pl.ds(g, 1), :], m)
        return carry

    lax.fori_loop(g0, g1 + 1, gbody, 0)

    @pl.when(pid == NBLK - 1)
    def _():
        pf = pooled[...]
        pf = jnp.where(jnp.isfinite(pf), pf, 0.0)
        out_ref[...] = (
            jnp.dot(pf, w2_ref[...], preferred_element_type=jnp.float32)
            + b2_ref[...])


_tc_call = pl.pallas_call(
    _tc_body,
    grid=(NBLK,),
    in_specs=[
        pl.BlockSpec((BN, D), lambda i: (i, 0)),
        pl.BlockSpec((2, BN, D), lambda i: (0, i, 0)),
        pl.BlockSpec((BN, NW), lambda i: (i, 0)),
        pl.BlockSpec((BN, 1), lambda i: (i, 0)),
        pl.BlockSpec((D, H), lambda i: (0, 0)),
        pl.BlockSpec((D, H), lambda i: (0, 0)),
        pl.BlockSpec((1, H), lambda i: (0, 0)),
        pl.BlockSpec((H, 1), lambda i: (0, 0)),
        pl.BlockSpec((1, 1), lambda i: (0, 0)),
    ],
    out_specs=pl.BlockSpec((G, 1), lambda i: (0, 0)),
    out_shape=jax.ShapeDtypeStruct((G, 1), jnp.float32),
    scratch_shapes=[pltpu.VMEM((G, H), jnp.float32)],
)


def kernel(x, edge_index, batch, W_l, b_l, W_r, W2, b2):
    pad = E_PAD - E
    edge_p = jnp.concatenate(
        [edge_index,
         jnp.stack([jnp.zeros((pad,), jnp.int32),
                    jnp.full((pad,), PAD_DST, jnp.int32)])], axis=1)
    dst16 = edge_p[1].reshape(E_PAD // 16, 16)
    edge_p = edge_p.reshape(2, E_PAD // CHUNK, CHUNK)

    agg2 = _sc_aggregate(edge_p, x)
    deg2 = _sc_degree(dst16)

    degT = jnp.transpose(deg2, (2, 0, 1)).reshape(N_PAD, NW)
    out = _tc_call(x, agg2, degT, batch.reshape(N, 1), W_l, W_r,
                   b_l.reshape(1, H), W2, b2.reshape(1, 1))
    return out.reshape(-1)


# speedup vs baseline: 12.3030x; 1.0835x over previous
import functools

import jax
import jax.numpy as jnp
from jax import lax
from jax.experimental import pallas as pl
from jax.experimental.pallas import tpu as pltpu
from jax.experimental.pallas import tpu_sc as plsc

N = 10000
E = 320000
D = 128
H = 128
G = 64

CHUNK = 128
NW = 32
CH_PER_W = 80
NCH = E // CHUNK
E_PAD = CH_PER_W * NW * CHUNK
N_PAD = 10112
ZROWS = N_PAD // 16
G_PER_W = E_PAD // 16 // NW
PAD_DST = 10008

_mesh = plsc.VectorSubcoreMesh(core_axis_name="c", subcore_axis_name="s")


@functools.partial(
    pl.kernel,
    mesh=_mesh,
    out_type=jax.ShapeDtypeStruct((2, N_PAD, D), jnp.float32),
    scratch_types=[
        pltpu.VMEM((CH_PER_W // 2, CHUNK), jnp.int32),
        pltpu.VMEM((CH_PER_W // 2, CHUNK), jnp.int32),
        pltpu.VMEM((2, CHUNK, D), jnp.float32),
        pltpu.VMEM_SHARED((N_PAD, D), jnp.float32),
        pltpu.SemaphoreType.DMA,
        pltpu.SemaphoreType.DMA,
    ],
)
def _sc_aggregate(edge_hbm, x_hbm, agg_out,
                  sidx, didx, rows, acc_sh, gsem, ssem):
    c = lax.axis_index("c")
    s = lax.axis_index("s")
    w = c * 16 + s

    def _z_body(i, carry):
        rows[0, i // 8, pl.ds((i % 8) * 16, 16)] = jnp.zeros((16,), jnp.float32)
        return carry

    lax.fori_loop(0, CHUNK * 8, _z_body, 0)
    z0 = s * ZROWS
    for t in range(4):
        pltpu.sync_copy(rows.at[0],
                        acc_sh.at[pl.ds(z0 + t * CHUNK, CHUNK)])
    pltpu.sync_copy(rows.at[0, pl.ds(0, ZROWS - 4 * CHUNK)],
                    acc_sh.at[pl.ds(z0 + 4 * CHUNK, ZROWS - 4 * CHUNK)])

    plsc.subcore_barrier()

    c0 = w * CH_PER_W
    nv = jnp.minimum(CH_PER_W, NCH - c0)
    HALF = CH_PER_W // 2

    for p in range(2):
        nvp = jnp.clip(nv - p * HALF, 0, HALF)

        @pl.when(nvp > 0)
        def _():
            pltpu.sync_copy(edge_hbm.at[0, pl.ds(c0 + p * HALF, HALF)], sidx)
            pltpu.sync_copy(edge_hbm.at[1, pl.ds(c0 + p * HALF, HALF)], didx)
            pltpu.async_copy(x_hbm.at[sidx.at[0]], rows.at[0], gsem)

            def chunk_body(i, carry):
                b = i % 2
                nb = 1 - b

                pltpu.make_async_copy(
                    x_hbm.at[pl.ds(0, CHUNK)], rows.at[b], gsem).wait()

                @pl.when(i >= 1)
                def _():
                    pltpu.make_async_copy(
                        x_hbm.at[pl.ds(0, CHUNK)], rows.at[nb], ssem).wait()

                @pl.when(i + 1 < nvp)
                def _():
                    pltpu.async_copy(
                        x_hbm.at[sidx.at[i + 1]], rows.at[nb], gsem)

                pltpu.async_copy(
                    rows.at[b], acc_sh.at[didx.at[i]], ssem, add=True)
                return carry

            lax.fori_loop(0, nvp, chunk_body, 0)
            pltpu.make_async_copy(
                x_hbm.at[pl.ds(0, CHUNK)], rows.at[0], ssem).wait()

    plsc.subcore_barrier()

    pltpu.sync_copy(acc_sh.at[pl.ds(z0, ZROWS)], agg_out.at[c, pl.ds(z0, ZROWS)])


@functools.partial(
    pl.kernel,
    mesh=_mesh,
    out_type=jax.ShapeDtypeStruct((2, 16, N_PAD), jnp.float32),
    scratch_types=[
        pltpu.VMEM((CH_PER_W, CHUNK), jnp.int32),
        pltpu.VMEM((N_PAD,), jnp.float32),
    ],
    compiler_params=pltpu.CompilerParams(needs_layout_passes=False),
)
def _sc_degree(edge_hbm, deg_out, didx, deg_l):
    c = lax.axis_index("c")
    s = lax.axis_index("s")
    w = c * 16 + s

    def _dz_body(i, carry):
        for k in range(8):
            deg_l[pl.ds((i * 8 + k) * 16, 16)] = jnp.zeros((16,), jnp.float32)
        return carry

    lax.fori_loop(0, N_PAD // 128, _dz_body, 0)

    pltpu.sync_copy(edge_hbm.at[1, pl.ds(w * CH_PER_W, CH_PER_W)], didx)

    ones = jnp.full((16,), 1.0, jnp.float32)

    def g_body(i, carry):
        for k in range(8):
            plsc.addupdate_scatter(
                deg_l, [didx[i, pl.ds(k * 16, 16)]], ones)
        return carry

    lax.fori_loop(0, CH_PER_W, g_body, 0)
    pltpu.sync_copy(deg_l, deg_out.at[c, s])


BN = 1000
NBLK = N // BN


def _tc_body(x_ref, a_ref, d_ref, b_ref, wl_ref, wr_ref, bl_ref, w2_ref,
             b2_ref, out_ref, pooled):
    pid = pl.program_id(0)

    @pl.when(pid == 0)
    def _():
        pooled[...] = jnp.full((G, H), -jnp.inf, jnp.float32)

    deg = jnp.sum(d_ref[...], axis=1, keepdims=True)
    mean = (a_ref[0] + a_ref[1]) / jnp.maximum(deg, 1.0)
    h = (jnp.dot(mean, wl_ref[...], preferred_element_type=jnp.float32)
         + jnp.dot(x_ref[...], wr_ref[...], preferred_element_type=jnp.float32)
         + bl_ref[...])
    h = jnp.maximum(h, 0.0)

    g0 = b_ref[0, 0]
    g1 = b_ref[BN - 1, 0]

    def gbody(g, carry):
        pen = jnp.where(b_ref[...] == g, 0.0, -jnp.inf)
        m = jnp.max(h + pen, axis=0, keepdims=True)
        pooled[pl.ds(g, 1), :] = jnp.maximum(pooled[pl.ds(g, 1), :], m)
        return carry

    lax.fori_loop(g0, g1 + 1, gbody, 0)

    @pl.when(pid == NBLK - 1)
    def _():
        pf = pooled[...]
        pf = jnp.where(jnp.isfinite(pf), pf, 0.0)
        out_ref[...] = (
            jnp.dot(pf, w2_ref[...], preferred_element_type=jnp.float32)
            + b2_ref[...])


_tc_call = pl.pallas_call(
    _tc_body,
    grid=(NBLK,),
    in_specs=[
        pl.BlockSpec((BN, D), lambda i: (i, 0)),
        pl.BlockSpec((2, BN, D), lambda i: (0, i, 0)),
        pl.BlockSpec((BN, NW), lambda i: (i, 0)),
        pl.BlockSpec((BN, 1), lambda i: (i, 0)),
        pl.BlockSpec((D, H), lambda i: (0, 0)),
        pl.BlockSpec((D, H), lambda i: (0, 0)),
        pl.BlockSpec((1, H), lambda i: (0, 0)),
        pl.BlockSpec((H, 1), lambda i: (0, 0)),
        pl.BlockSpec((1, 1), lambda i: (0, 0)),
    ],
    out_specs=pl.BlockSpec((G, 1), lambda i: (0, 0)),
    out_shape=jax.ShapeDtypeStruct((G, 1), jnp.float32),
    scratch_shapes=[pltpu.VMEM((G, H), jnp.float32)],
)


def kernel(x, edge_index, batch, W_l, b_l, W_r, W2, b2):
    pad = E_PAD - E
    edge_p = jnp.concatenate(
        [edge_index,
         jnp.stack([jnp.zeros((pad,), jnp.int32),
                    jnp.full((pad,), PAD_DST, jnp.int32)])], axis=1)
    edge_p = edge_p.reshape(2, E_PAD // CHUNK, CHUNK)

    agg2 = _sc_aggregate(edge_p, x)
    deg2 = _sc_degree(edge_p)

    degT = jnp.transpose(deg2, (2, 0, 1)).reshape(N_PAD, NW)
    out = _tc_call(x, agg2, degT, batch.reshape(N, 1), W_l, W_r,
                   b_l.reshape(1, H), W2, b2.reshape(1, 1))
    return out.reshape(-1)


# speedup vs baseline: 12.3082x; 1.0004x over previous
import functools

import jax
import jax.numpy as jnp
from jax import lax
from jax.experimental import pallas as pl
from jax.experimental.pallas import tpu as pltpu
from jax.experimental.pallas import tpu_sc as plsc

N = 10000
E = 320000
D = 128
H = 128
G = 64

CHUNK = 128
NW = 32
CH_PER_W = 80
NCH = E // CHUNK
E_PAD = CH_PER_W * NW * CHUNK
N_PAD = 10112
ZROWS = N_PAD // 16
G_PER_W = E_PAD // 16 // NW
PAD_DST = 10008

_mesh = plsc.VectorSubcoreMesh(core_axis_name="c", subcore_axis_name="s")


@functools.partial(
    pl.kernel,
    mesh=_mesh,
    out_type=jax.ShapeDtypeStruct((2, N_PAD, D), jnp.float32),
    scratch_types=[
        pltpu.VMEM((CH_PER_W // 2, CHUNK), jnp.int32),
        pltpu.VMEM((CH_PER_W // 2, CHUNK), jnp.int32),
        pltpu.VMEM((2, CHUNK, D), jnp.float32),
        pltpu.VMEM_SHARED((N_PAD, D), jnp.float32),
        pltpu.SemaphoreType.DMA,
        pltpu.SemaphoreType.DMA,
    ],
)
def _sc_aggregate(edge_hbm, x_hbm, agg_out,
                  sidx, didx, rows, acc_sh, gsem, ssem):
    c = lax.axis_index("c")
    s = lax.axis_index("s")
    w = c * 16 + s

    def _z_body(i, carry):
        rows[0, i // 8, pl.ds((i % 8) * 16, 16)] = jnp.zeros((16,), jnp.float32)
        return carry

    lax.fori_loop(0, CHUNK * 8, _z_body, 0)
    z0 = s * ZROWS
    for t in range(4):
        pltpu.sync_copy(rows.at[0],
                        acc_sh.at[pl.ds(z0 + t * CHUNK, CHUNK)])
    pltpu.sync_copy(rows.at[0, pl.ds(0, ZROWS - 4 * CHUNK)],
                    acc_sh.at[pl.ds(z0 + 4 * CHUNK, ZROWS - 4 * CHUNK)])

    plsc.subcore_barrier()

    c0 = w * CH_PER_W
    nv = jnp.minimum(CH_PER_W, NCH - c0)
    HALF = CH_PER_W // 2

    for p in range(2):
        nvp = jnp.clip(nv - p * HALF, 0, HALF)

        @pl.when(nvp > 0)
        def _():
            pltpu.sync_copy(edge_hbm.at[0, pl.ds(c0 + p * HALF, HALF)], sidx)
            pltpu.sync_copy(edge_hbm.at[1, pl.ds(c0 + p * HALF, HALF)], didx)
            pltpu.async_copy(x_hbm.at[sidx.at[0]], rows.at[0], gsem)

            def chunk_body(i, carry):
                b = i % 2
                nb = 1 - b

                pltpu.make_async_copy(
                    x_hbm.at[pl.ds(0, CHUNK)], rows.at[b], gsem).wait()

                @pl.when(i >= 1)
                def _():
                    pltpu.make_async_copy(
                        x_hbm.at[pl.ds(0, CHUNK)], rows.at[nb], ssem).wait()

                @pl.when(i + 1 < nvp)
                def _():
                    pltpu.async_copy(
                        x_hbm.at[sidx.at[i + 1]], rows.at[nb], gsem)

                pltpu.async_copy(
                    rows.at[b], acc_sh.at[didx.at[i]], ssem, add=True)
                return carry

            lax.fori_loop(0, nvp, chunk_body, 0)
            pltpu.make_async_copy(
                x_hbm.at[pl.ds(0, CHUNK)], rows.at[0], ssem).wait()

    plsc.subcore_barrier()

    pltpu.sync_copy(acc_sh.at[pl.ds(z0, ZROWS)], agg_out.at[c, pl.ds(z0, ZROWS)])


@functools.partial(
    pl.kernel,
    mesh=_mesh,
    out_type=jax.ShapeDtypeStruct((2, 16, N_PAD), jnp.float32),
    scratch_types=[
        pltpu.VMEM((CH_PER_W, CHUNK), jnp.int32),
        pltpu.VMEM((N_PAD,), jnp.float32),
    ],
    compiler_params=pltpu.CompilerParams(needs_layout_passes=False),
)
def _sc_degree(edge_hbm, deg_out, didx, deg_l):
    c = lax.axis_index("c")
    s = lax.axis_index("s")
    w = c * 16 + s

    def _dz_body(i, carry):
        for k in range(8):
            deg_l[pl.ds((i * 8 + k) * 16, 16)] = jnp.zeros((16,), jnp.float32)
        return carry

    lax.fori_loop(0, N_PAD // 128, _dz_body, 0)

    pltpu.sync_copy(edge_hbm.at[1, pl.ds(w * CH_PER_W, CH_PER_W)], didx)

    ones = jnp.full((16,), 1.0, jnp.float32)

    def g_body(i, carry):
        for k in range(8):
            plsc.addupdate_scatter(
                deg_l, [didx[i, pl.ds(k * 16, 16)]], ones)
        return carry

    lax.fori_loop(0, CH_PER_W, g_body, 0)
    pltpu.sync_copy(deg_l, deg_out.at[c, s])


BN = 1000
NBLK = N // BN


def _tc_body(x_ref, a_ref, d_ref, b_ref, wl_ref, wr_ref, bl_ref, w2_ref,
             b2_ref, out_ref, pooled, dscr):
    pid = pl.program_id(0)

    @pl.when(pid == 0)
    def _():
        pooled[...] = jnp.full((G, H), -jnp.inf, jnp.float32)
        dscr[...] = jnp.transpose(d_ref[...].reshape(NW, N_PAD))

    deg = jnp.sum(dscr[pl.ds(pid * BN, BN), :], axis=1,
                  keepdims=True)
    mean = (a_ref[0] + a_ref[1]) / jnp.maximum(deg, 1.0)
    h = (jnp.dot(mean, wl_ref[...], preferred_element_type=jnp.float32)
         + jnp.dot(x_ref[...], wr_ref[...], preferred_element_type=jnp.float32)
         + bl_ref[...])
    h = jnp.maximum(h, 0.0)

    g0 = b_ref[0, 0]
    g1 = b_ref[BN - 1, 0]

    def gbody(g, carry):
        pen = jnp.where(b_ref[...] == g, 0.0, -jnp.inf)
        m = jnp.max(h + pen, axis=0, keepdims=True)
        pooled[pl.ds(g, 1), :] = jnp.maximum(pooled[pl.ds(g, 1), :], m)
        return carry

    lax.fori_loop(g0, g1 + 1, gbody, 0)

    @pl.when(pid == NBLK - 1)
    def _():
        pf = pooled[...]
        pf = jnp.where(jnp.isfinite(pf), pf, 0.0)
        out_ref[...] = (
            jnp.dot(pf, w2_ref[...], preferred_element_type=jnp.float32)
            + b2_ref[...])


_tc_call = pl.pallas_call(
    _tc_body,
    grid=(NBLK,),
    in_specs=[
        pl.BlockSpec((BN, D), lambda i: (i, 0)),
        pl.BlockSpec((2, BN, D), lambda i: (0, i, 0)),
        pl.BlockSpec((2, 16, N_PAD), lambda i: (0, 0, 0)),
        pl.BlockSpec((BN, 1), lambda i: (i, 0)),
        pl.BlockSpec((D, H), lambda i: (0, 0)),
        pl.BlockSpec((D, H), lambda i: (0, 0)),
        pl.BlockSpec((1, H), lambda i: (0, 0)),
        pl.BlockSpec((H, 1), lambda i: (0, 0)),
        pl.BlockSpec((1, 1), lambda i: (0, 0)),
    ],
    out_specs=pl.BlockSpec((G, 1), lambda i: (0, 0)),
    out_shape=jax.ShapeDtypeStruct((G, 1), jnp.float32),
    scratch_shapes=[pltpu.VMEM((G, H), jnp.float32),
                    pltpu.VMEM((N_PAD, NW), jnp.float32)],
)


def kernel(x, edge_index, batch, W_l, b_l, W_r, W2, b2):
    pad = E_PAD - E
    edge_p = jnp.concatenate(
        [edge_index,
         jnp.stack([jnp.zeros((pad,), jnp.int32),
                    jnp.full((pad,), PAD_DST, jnp.int32)])], axis=1)
    edge_p = edge_p.reshape(2, E_PAD // CHUNK, CHUNK)

    agg2 = _sc_aggregate(edge_p, x)
    deg2 = _sc_degree(edge_p)

    out = _tc_call(x, agg2, deg2, batch.reshape(N, 1), W_l, W_r,
                   b_l.reshape(1, H), W2, b2.reshape(1, 1))
    return out.reshape(-1)


# speedup vs baseline: 12.3942x; 1.0070x over previous
import functools

import jax
import jax.numpy as jnp
from jax import lax
from jax.experimental import pallas as pl
from jax.experimental.pallas import tpu as pltpu
from jax.experimental.pallas import tpu_sc as plsc

N = 10000
E = 320000
D = 128
H = 128
G = 64

CHUNK = 128
NW = 32
CH_PER_W = 80
NCH = E // CHUNK
E_PAD = CH_PER_W * NW * CHUNK
N_PAD = 10112
ZROWS = N_PAD // 16
G_PER_W = E_PAD // 16 // NW
PAD_DST = 10008

_mesh = plsc.VectorSubcoreMesh(core_axis_name="c", subcore_axis_name="s")


@functools.partial(
    pl.kernel,
    mesh=_mesh,
    out_type=jax.ShapeDtypeStruct((2, N_PAD, D), jnp.float32),
    scratch_types=[
        pltpu.VMEM((CH_PER_W // 2, CHUNK), jnp.int32),
        pltpu.VMEM((CH_PER_W // 2, CHUNK), jnp.int32),
        pltpu.VMEM((2, CHUNK, D), jnp.float32),
        pltpu.VMEM_SHARED((N_PAD, D), jnp.float32),
        pltpu.SemaphoreType.DMA,
        pltpu.SemaphoreType.DMA,
    ],
)
def _sc_aggregate(edge_hbm, x_hbm, agg_out,
                  sidx, didx, rows, acc_sh, gsem, ssem):
    c = lax.axis_index("c")
    s = lax.axis_index("s")
    w = c * 16 + s

    def _z_body(i, carry):
        rows[0, i // 8, pl.ds((i % 8) * 16, 16)] = jnp.zeros((16,), jnp.float32)
        return carry

    lax.fori_loop(0, CHUNK * 8, _z_body, 0)
    z0 = s * ZROWS
    for t in range(4):
        pltpu.async_copy(rows.at[0],
                         acc_sh.at[pl.ds(z0 + t * CHUNK, CHUNK)], ssem)
    pltpu.async_copy(rows.at[0, pl.ds(0, ZROWS - 4 * CHUNK)],
                     acc_sh.at[pl.ds(z0 + 4 * CHUNK, ZROWS - 4 * CHUNK)],
                     ssem)

    c0 = w * CH_PER_W
    nv = jnp.minimum(CH_PER_W, NCH - c0)
    HALF = CH_PER_W // 2

    for p in range(2):
        nvp = jnp.clip(nv - p * HALF, 0, HALF)

        @pl.when(nvp > 0)
        def _():
            pltpu.sync_copy(edge_hbm.at[0, pl.ds(c0 + p * HALF, HALF)], sidx)
            pltpu.sync_copy(edge_hbm.at[1, pl.ds(c0 + p * HALF, HALF)], didx)
            if p == 0:
                for t in range(4):
                    pltpu.make_async_copy(
                        x_hbm.at[pl.ds(0, CHUNK)], rows.at[0], ssem).wait()
                pltpu.make_async_copy(
                    x_hbm.at[pl.ds(0, ZROWS - 4 * CHUNK)],
                    rows.at[0, pl.ds(0, ZROWS - 4 * CHUNK)], ssem).wait()
                plsc.subcore_barrier()
            pltpu.async_copy(x_hbm.at[sidx.at[0]], rows.at[0], gsem)

            def chunk_body(i, carry):
                b = i % 2
                nb = 1 - b

                pltpu.make_async_copy(
                    x_hbm.at[pl.ds(0, CHUNK)], rows.at[b], gsem).wait()

                @pl.when(i >= 1)
                def _():
                    pltpu.make_async_copy(
                        x_hbm.at[pl.ds(0, CHUNK)], rows.at[nb], ssem).wait()

                @pl.when(i + 1 < nvp)
                def _():
                    pltpu.async_copy(
                        x_hbm.at[sidx.at[i + 1]], rows.at[nb], gsem)

                pltpu.async_copy(
                    rows.at[b], acc_sh.at[didx.at[i]], ssem, add=True)
                return carry

            lax.fori_loop(0, nvp, chunk_body, 0)
            pltpu.make_async_copy(
                x_hbm.at[pl.ds(0, CHUNK)], rows.at[0], ssem).wait()

    plsc.subcore_barrier()

    pltpu.sync_copy(acc_sh.at[pl.ds(z0, ZROWS)], agg_out.at[c, pl.ds(z0, ZROWS)])


@functools.partial(
    pl.kernel,
    mesh=_mesh,
    out_type=jax.ShapeDtypeStruct((2, 16, N_PAD), jnp.float32),
    scratch_types=[
        pltpu.VMEM((CH_PER_W, CHUNK), jnp.int32),
        pltpu.VMEM((N_PAD,), jnp.float32),
    ],
    compiler_params=pltpu.CompilerParams(needs_layout_passes=False),
)
def _sc_degree(edge_hbm, deg_out, didx, deg_l):
    c = lax.axis_index("c")
    s = lax.axis_index("s")
    w = c * 16 + s

    def _dz_body(i, carry):
        for k in range(8):
            deg_l[pl.ds((i * 8 + k) * 16, 16)] = jnp.zeros((16,), jnp.float32)
        return carry

    lax.fori_loop(0, N_PAD // 128, _dz_body, 0)

    pltpu.sync_copy(edge_hbm.at[1, pl.ds(w * CH_PER_W, CH_PER_W)], didx)

    ones = jnp.full((16,), 1.0, jnp.float32)

    def g_body(i, carry):
        for k in range(8):
            plsc.addupdate_scatter(
                deg_l, [didx[i, pl.ds(k * 16, 16)]], ones)
        return carry

    lax.fori_loop(0, CH_PER_W, g_body, 0)
    pltpu.sync_copy(deg_l, deg_out.at[c, s])


BN = 1000
NBLK = N // BN


def _tc_body(x_ref, a_ref, d_ref, b_ref, wl_ref, wr_ref, bl_ref, w2_ref,
             b2_ref, out_ref, pooled, dscr):
    pid = pl.program_id(0)

    @pl.when(pid == 0)
    def _():
        pooled[...] = jnp.full((G, H), -jnp.inf, jnp.float32)
        dscr[...] = jnp.transpose(d_ref[...].reshape(NW, N_PAD))

    deg = jnp.sum(dscr[pl.ds(pid * BN, BN), :], axis=1,
                  keepdims=True)
    mean = (a_ref[0] + a_ref[1]) / jnp.maximum(deg, 1.0)
    h = (jnp.dot(mean, wl_ref[...], preferred_element_type=jnp.float32)
         + jnp.dot(x_ref[...], wr_ref[...], preferred_element_type=jnp.float32)
         + bl_ref[...])
    h = jnp.maximum(h, 0.0)

    g0 = b_ref[0, 0]
    g1 = b_ref[BN - 1, 0]

    def gbody(g, carry):
        pen = jnp.where(b_ref[...] == g, 0.0, -jnp.inf)
        m = jnp.max(h + pen, axis=0, keepdims=True)
        pooled[pl.ds(g, 1), :] = jnp.maximum(pooled[pl.ds(g, 1), :], m)
        return carry

    lax.fori_loop(g0, g1 + 1, gbody, 0)

    @pl.when(pid == NBLK - 1)
    def _():
        pf = pooled[...]
        pf = jnp.where(jnp.isfinite(pf), pf, 0.0)
        out_ref[...] = (
            jnp.dot(pf, w2_ref[...], preferred_element_type=jnp.float32)
            + b2_ref[...])


_tc_call = pl.pallas_call(
    _tc_body,
    grid=(NBLK,),
    in_specs=[
        pl.BlockSpec((BN, D), lambda i: (i, 0)),
        pl.BlockSpec((2, BN, D), lambda i: (0, i, 0)),
        pl.BlockSpec((2, 16, N_PAD), lambda i: (0, 0, 0)),
        pl.BlockSpec((BN, 1), lambda i: (i, 0)),
        pl.BlockSpec((D, H), lambda i: (0, 0)),
        pl.BlockSpec((D, H), lambda i: (0, 0)),
        pl.BlockSpec((1, H), lambda i: (0, 0)),
        pl.BlockSpec((H, 1), lambda i: (0, 0)),
        pl.BlockSpec((1, 1), lambda i: (0, 0)),
    ],
    out_specs=pl.BlockSpec((G, 1), lambda i: (0, 0)),
    out_shape=jax.ShapeDtypeStruct((G, 1), jnp.float32),
    scratch_shapes=[pltpu.VMEM((G, H), jnp.float32),
                    pltpu.VMEM((N_PAD, NW), jnp.float32)],
)


def kernel(x, edge_index, batch, W_l, b_l, W_r, W2, b2):
    pad = E_PAD - E
    edge_p = jnp.concatenate(
        [edge_index,
         jnp.stack([jnp.zeros((pad,), jnp.int32),
                    jnp.full((pad,), PAD_DST, jnp.int32)])], axis=1)
    edge_p = edge_p.reshape(2, E_PAD // CHUNK, CHUNK)

    agg2 = _sc_aggregate(edge_p, x)
    deg2 = _sc_degree(edge_p)

    out = _tc_call(x, agg2, deg2, batch.reshape(N, 1), W_l, W_r,
                   b_l.reshape(1, H), W2, b2.reshape(1, 1))
    return out.reshape(-1)
